# baseline (speedup 1.0000x reference)
import jax
import jax.numpy as jnp
from jax.experimental import pallas as pl
from jax.experimental.pallas import tpu as pltpu

N_EXP = 16
TOPK = 2
EPS = 0.05
ITERS = 20
BM = 1024


def _router_kernel(xa_ref, xb_ref, w_ref, pit_ref, idx_ref, loss_ref, k_ref, b_ref):
    t = pl.program_id(0)
    nt = pl.num_programs(0)
    w = w_ref[...]
    for half, xr in enumerate((xa_ref, xb_ref)):
        lg = jax.lax.dot_general(
            w, xr[...], (((1,), (1,)), ((), ())), preferred_element_type=jnp.float32
        )
        la = lg * (-1.0 / EPS)
        m = jnp.max(la, axis=0, keepdims=True)
        kt = jnp.exp(la - m)
        off = (2 * t + half) * BM
        k_ref[:, pl.ds(off, BM)] = kt
        b_ref[:, pl.ds(off, BM)] = 1.0 / jnp.sum(kt, axis=0, keepdims=True)

    @pl.when(t == nt - 1)
    def _finalize():
        scale = float(pit_ref.shape[1]) / N_EXP

        def body(_, ab):
            a, b = ab
            a = scale / jnp.sum(k_ref[...] * b, axis=1, keepdims=True)
            b = 1.0 / jnp.sum(k_ref[...] * a, axis=0, keepdims=True)
            return (a, b)

        a0 = jnp.zeros((N_EXP, 1), jnp.float32)
        b0 = b_ref[...]
        a, b = jax.lax.fori_loop(0, ITERS - 1, body, (a0, b0))
        a = 1.0 / jnp.sum(k_ref[...] * b, axis=1, keepdims=True)
        pit = k_ref[...] * a * b
        pit_ref[...] = pit

        iota = jax.lax.broadcasted_iota(jnp.int32, pit.shape, 0)
        mx1 = jnp.max(pit, axis=0, keepdims=True)
        i1 = jnp.min(jnp.where(pit == mx1, iota, N_EXP), axis=0, keepdims=True)
        masked = jnp.where(iota == i1, -jnp.inf, pit)
        mx2 = jnp.max(masked, axis=0, keepdims=True)
        i2 = jnp.min(jnp.where(masked == mx2, iota, N_EXP), axis=0, keepdims=True)
        idx_ref[0:1, :] = i1
        idx_ref[1:2, :] = i2

        u = 1.0 / N_EXP
        load = jnp.mean(pit, axis=1, keepdims=True)
        loss_ref[...] = jnp.sum(
            u * (jnp.log(u) - jnp.log(load)), axis=(0, 1), keepdims=True
        )


def kernel(x, gate_w, centroids):
    b, t, d = x.shape
    n = b * t
    x2 = x.reshape(n, d)
    pit, idxt, loss = pl.pallas_call(
        _router_kernel,
        grid=(n // (2 * BM),),
        in_specs=[
            pl.BlockSpec((BM, d), lambda i: (2 * i, 0)),
            pl.BlockSpec((BM, d), lambda i: (2 * i + 1, 0)),
            pl.BlockSpec((N_EXP, d), lambda i: (0, 0)),
        ],
        out_specs=[
            pl.BlockSpec((N_EXP, n), lambda i: (0, 0)),
            pl.BlockSpec((TOPK, n), lambda i: (0, 0)),
            pl.BlockSpec((1, 1), lambda i: (0, 0)),
        ],
        out_shape=[
            jax.ShapeDtypeStruct((N_EXP, n), jnp.float32),
            jax.ShapeDtypeStruct((TOPK, n), jnp.int32),
            jax.ShapeDtypeStruct((1, 1), jnp.float32),
        ],
        scratch_shapes=[
            pltpu.VMEM((N_EXP, n), jnp.float32),
            pltpu.VMEM((1, n), jnp.float32),
        ],
    )(x2, x2, gate_w)
    dispatch = pit.T.reshape(b, t, N_EXP)
    indices = idxt.T.reshape(b, t, TOPK)
    load_loss = loss[0, 0]
    return dispatch, indices, load_loss


# speedup vs baseline: 1.0927x; 1.0927x over previous
import jax
import jax.numpy as jnp
from jax.experimental import pallas as pl
from jax.experimental.pallas import tpu as pltpu

N_EXP = 16
TOPK = 2
EPS = 0.05
ITERS = 20
BM = 1024


def _router_kernel(x_ref, w_ref, pit_ref, idx_ref, loss_ref, k_ref, b_ref):
    t = pl.program_id(0)
    nt = pl.num_programs(0)
    xb = x_ref[...]
    w = w_ref[...]
    lg = jax.lax.dot_general(
        w, xb, (((1,), (1,)), ((), ())), preferred_element_type=jnp.float32
    )
    la = lg * (-1.0 / EPS)
    m = jnp.max(la, axis=0, keepdims=True)
    kt = jnp.exp(la - m)
    k_ref[:, pl.ds(t * BM, BM)] = kt
    b_ref[:, pl.ds(t * BM, BM)] = 1.0 / jnp.sum(kt, axis=0, keepdims=True)

    @pl.when(t == nt - 1)
    def _finalize():
        scale = float(pit_ref.shape[1]) / N_EXP

        b = b_ref[...]
        for _ in range(ITERS - 1):
            k = k_ref[...]
            a = scale / jnp.sum(k * b, axis=1, keepdims=True)
            b = 1.0 / jnp.sum(k * a, axis=0, keepdims=True)
        k = k_ref[...]
        a = 1.0 / jnp.sum(k * b, axis=1, keepdims=True)
        pit = k * a * b
        pit_ref[...] = pit

        iota = jax.lax.broadcasted_iota(jnp.int32, pit.shape, 0)
        mx1 = jnp.max(pit, axis=0, keepdims=True)
        i1 = jnp.min(jnp.where(pit == mx1, iota, N_EXP), axis=0, keepdims=True)
        masked = jnp.where(iota == i1, -jnp.inf, pit)
        mx2 = jnp.max(masked, axis=0, keepdims=True)
        i2 = jnp.min(jnp.where(masked == mx2, iota, N_EXP), axis=0, keepdims=True)
        idx_ref[0:1, :] = i1
        idx_ref[1:2, :] = i2

        u = 1.0 / N_EXP
        load = jnp.mean(pit, axis=1, keepdims=True)
        loss_ref[...] = jnp.sum(
            u * (jnp.log(u) - jnp.log(load)), axis=(0, 1), keepdims=True
        )


def kernel(x, gate_w, centroids):
    b, t, d = x.shape
    n = b * t
    x2 = x.reshape(n, d)
    pit, idxt, loss = pl.pallas_call(
        _router_kernel,
        grid=(n // BM,),
        in_specs=[
            pl.BlockSpec((BM, d), lambda i: (i, 0)),
            pl.BlockSpec((N_EXP, d), lambda i: (0, 0)),
        ],
        out_specs=[
            pl.BlockSpec((N_EXP, n), lambda i: (0, 0)),
            pl.BlockSpec((TOPK, n), lambda i: (0, 0)),
            pl.BlockSpec((1, 1), lambda i: (0, 0)),
        ],
        out_shape=[
            jax.ShapeDtypeStruct((N_EXP, n), jnp.float32),
            jax.ShapeDtypeStruct((TOPK, n), jnp.int32),
            jax.ShapeDtypeStruct((1, 1), jnp.float32),
        ],
        scratch_shapes=[
            pltpu.VMEM((N_EXP, n), jnp.float32),
            pltpu.VMEM((1, n), jnp.float32),
        ],
    )(x2, gate_w)
    dispatch = pit.T.reshape(b, t, N_EXP)
    indices = idxt.T.reshape(b, t, TOPK)
    load_loss = loss[0, 0]
    return dispatch, indices, load_loss


# speedup vs baseline: 1.0975x; 1.0044x over previous
import jax
import jax.numpy as jnp
from jax.experimental import pallas as pl
from jax.experimental.pallas import tpu as pltpu

N_EXP = 16
TOPK = 2
EPS = 0.05
ITERS = 20
BM = 512


def _router_kernel(xa_ref, xb_ref, w_ref, pit_ref, idx_ref, loss_ref, k_ref, b_ref):
    t = pl.program_id(0)
    nt = pl.num_programs(0)
    w = w_ref[...]
    for half, xr in enumerate((xa_ref, xb_ref)):
        lg = jax.lax.dot_general(
            w, xr[...], (((1,), (1,)), ((), ())), preferred_element_type=jnp.float32
        )
        la = lg * (-1.0 / EPS)
        m = jnp.max(la, axis=0, keepdims=True)
        kt = jnp.exp(la - m)
        off = (2 * t + half) * BM
        k_ref[:, pl.ds(off, BM)] = kt
        b_ref[:, pl.ds(off, BM)] = 1.0 / jnp.sum(kt, axis=0, keepdims=True)

    @pl.when(t == nt - 1)
    def _finalize():
        scale = float(pit_ref.shape[1]) / N_EXP

        b = b_ref[...]
        for _ in range(ITERS - 1):
            k = k_ref[...]
            a = scale / jnp.sum(k * b, axis=1, keepdims=True)
            b = 1.0 / jnp.sum(k * a, axis=0, keepdims=True)
        k = k_ref[...]
        a = 1.0 / jnp.sum(k * b, axis=1, keepdims=True)
        pit = k * a * b
        pit_ref[...] = pit

        iota = jax.lax.broadcasted_iota(jnp.int32, pit.shape, 0)
        mx1 = jnp.max(pit, axis=0, keepdims=True)
        i1 = jnp.min(jnp.where(pit == mx1, iota, N_EXP), axis=0, keepdims=True)
        masked = jnp.where(iota == i1, -jnp.inf, pit)
        mx2 = jnp.max(masked, axis=0, keepdims=True)
        i2 = jnp.min(jnp.where(masked == mx2, iota, N_EXP), axis=0, keepdims=True)
        idx_ref[0:1, :] = i1
        idx_ref[1:2, :] = i2

        u = 1.0 / N_EXP
        load = jnp.mean(pit, axis=1, keepdims=True)
        loss_ref[...] = jnp.sum(
            u * (jnp.log(u) - jnp.log(load)), axis=(0, 1), keepdims=True
        )


def kernel(x, gate_w, centroids):
    b, t, d = x.shape
    n = b * t
    x2 = x.reshape(n, d)
    pit, idxt, loss = pl.pallas_call(
        _router_kernel,
        grid=(n // (2 * BM),),
        in_specs=[
            pl.BlockSpec((BM, d), lambda i: (2 * i, 0)),
            pl.BlockSpec((BM, d), lambda i: (2 * i + 1, 0)),
            pl.BlockSpec((N_EXP, d), lambda i: (0, 0)),
        ],
        out_specs=[
            pl.BlockSpec((N_EXP, n), lambda i: (0, 0)),
            pl.BlockSpec((TOPK, n), lambda i: (0, 0)),
            pl.BlockSpec((1, 1), lambda i: (0, 0)),
        ],
        out_shape=[
            jax.ShapeDtypeStruct((N_EXP, n), jnp.float32),
            jax.ShapeDtypeStruct((TOPK, n), jnp.int32),
            jax.ShapeDtypeStruct((1, 1), jnp.float32),
        ],
        scratch_shapes=[
            pltpu.VMEM((N_EXP, n), jnp.float32),
            pltpu.VMEM((1, n), jnp.float32),
        ],
    )(x2, x2, gate_w)
    dispatch = pit.T.reshape(b, t, N_EXP)
    indices = idxt.T.reshape(b, t, TOPK)
    load_loss = loss[0, 0]
    return dispatch, indices, load_loss
